# Initial kernel scaffold; baseline (speedup 1.0000x reference)
import jax
import jax.numpy as jnp
from jax.experimental import pallas as pl


def kernel(w0, b0, w1, b1, w2, b2, mlp_w1, mlp_b1, mlp_w2, mlp_b2):
    raise NotImplementedError("write your pallas kernel here")


# speedup vs baseline: 4.4279x; 4.4279x over previous
import functools

import numpy as np
import jax
import jax.numpy as jnp
from jax.experimental import pallas as pl

_I32_MIN = np.int32(-2147483648)
_I32_MAX = np.int32(2147483647)


def _f32_to_key(b):
    return b ^ ((b >> 31) & jnp.int32(0x7FFFFFFF))


def _key_to_f32(k):
    b = k ^ ((k >> 31) & jnp.int32(0x7FFFFFFF))
    return jax.lax.bitcast_convert_type(b, jnp.float32)


def _stats_body(x_ref, o_ref, *, n, base_ranks, fracs):
    x = x_ref[...]
    fn = jnp.float32(n)
    mean = jnp.sum(x, axis=1, keepdims=True) / fn
    d = x - mean
    var = jnp.sum(d * d, axis=1, keepdims=True) / jnp.float32(n - 1)
    xmin = jnp.min(x, axis=1, keepdims=True)
    xmax = jnp.max(x, axis=1, keepdims=True)

    b = jax.lax.bitcast_convert_type(x, jnp.int32)
    skey = _f32_to_key(b)
    ukey = skey ^ _I32_MIN

    quants = []
    for k, frac in zip(base_ranks, fracs):
        p = jnp.zeros((x.shape[0], 1), jnp.int32)
        rem = jnp.full((x.shape[0], 1), k, jnp.int32)
        for bit in range(31, -1, -1):
            m = jnp.int32(np.uint32((0xFFFFFFFF << bit) & 0xFFFFFFFF).view(np.int32))
            w = ukey & m
            c0 = jnp.sum(jnp.where(w == p, 1, 0), axis=1, keepdims=True)
            take1 = rem >= c0
            bitv = jnp.int32(np.uint32(1 << bit).view(np.int32))
            p = jnp.where(take1, p | bitv, p)
            rem = jnp.where(take1, rem - c0, rem)
        sp = p ^ _I32_MIN
        v_lo = _key_to_f32(sp)
        cnt_le = jnp.sum(jnp.where(skey <= sp, 1, 0), axis=1, keepdims=True)
        nxt = jnp.min(jnp.where(skey > sp, skey, _I32_MAX), axis=1, keepdims=True)
        v_hi = jnp.where(cnt_le >= k + 2, v_lo, _key_to_f32(nxt))
        quants.append(v_lo * (1.0 - frac) + v_hi * frac)

    zero = jnp.zeros_like(mean)
    o_ref[...] = jnp.concatenate(
        [mean, var, xmin, quants[0], quants[1], quants[2], xmax, zero], axis=1)


def _stats(x, row_block):
    B, n = x.shape
    base_ranks = [int(np.floor(q * (n - 1))) for q in (0.25, 0.5, 0.75)]
    fracs = [float(np.float32(q * (n - 1) - np.floor(q * (n - 1))))
             for q in (0.25, 0.5, 0.75)]
    body = functools.partial(_stats_body, n=n, base_ranks=base_ranks, fracs=fracs)
    return pl.pallas_call(
        body,
        grid=(B // row_block,),
        in_specs=[pl.BlockSpec((row_block, n), lambda i: (i, 0))],
        out_specs=pl.BlockSpec((row_block, 8), lambda i: (i, 0)),
        out_shape=jax.ShapeDtypeStruct((B, 8), jnp.float32),
    )(x)


def _mlp_body(x_ref, w1_ref, b1_ref, w2_ref, b2_ref, o_ref):
    x = x_ref[...]
    h = jnp.dot(x, w1_ref[...], preferred_element_type=jnp.float32,
                precision=jax.lax.Precision.HIGHEST) + b1_ref[...]
    h = jnp.maximum(h, 0.0)
    o_ref[...] = jnp.dot(h, w2_ref[...], preferred_element_type=jnp.float32,
                         precision=jax.lax.Precision.HIGHEST) + b2_ref[...]


def _mlp(feats, w1t, b1, w2t, b2, row_block=512):
    B, F = feats.shape
    H1 = w1t.shape[1]
    H2 = w2t.shape[1]
    return pl.pallas_call(
        _mlp_body,
        grid=(B // row_block,),
        in_specs=[
            pl.BlockSpec((row_block, F), lambda i: (i, 0)),
            pl.BlockSpec((F, H1), lambda i: (0, 0)),
            pl.BlockSpec((1, H1), lambda i: (0, 0)),
            pl.BlockSpec((H1, H2), lambda i: (0, 0)),
            pl.BlockSpec((1, H2), lambda i: (0, 0)),
        ],
        out_specs=pl.BlockSpec((row_block, H2), lambda i: (i, 0)),
        out_shape=jax.ShapeDtypeStruct((B, H2), jnp.float32),
    )(feats, w1t, b1, w2t, b2)


def kernel(w0, b0, w1, b1, w2, b2, mlp_w1, mlp_b1, mlp_w2, mlp_b2):
    B = w0.shape[0]
    params = [w0, b0, w1, b1, w2, b2]
    feats = []
    for p in params:
        p2 = p.reshape(B, -1)
        rb = 128 if p2.shape[1] >= 1024 else 512
        feats.append(_stats(p2, rb))
    feats48 = jnp.concatenate(feats, axis=1)

    w1p = jnp.pad(mlp_w1.reshape(mlp_w1.shape[0], 6, 7),
                  ((0, 0), (0, 0), (0, 1))).reshape(mlp_w1.shape[0], 48)
    out = _mlp(feats48, w1p.T, mlp_b1.reshape(1, -1), mlp_w2.T,
               mlp_b2.reshape(1, -1))
    return out
